# baseline (speedup 1.0000x reference)
import functools

import jax
import jax.numpy as jnp
from jax import lax
from jax.experimental import pallas as pl
from jax.experimental.pallas import tpu as pltpu
from jax.experimental.pallas import tpu_sc as plsc

N = 10000
F = 128
D = 64
C = 3
E = 160000

NC, NS, L = 2, 16, 16
NW = NC * NS
CH = 128
NCH = 40
EPT = NCH * CH
E_PAD = NW * EPT
TRASH = N
R = 10240
RPT = R // NS

BLK = 512
_mesh = plsc.VectorSubcoreMesh(core_axis_name="c", subcore_axis_name="s")
_sc_params = pltpu.CompilerParams(use_tc_tiling_on_sc=False)


DW = 16


@functools.partial(
    pl.kernel,
    out_type=jax.ShapeDtypeStruct((NC, R, DW), jnp.float32),
    mesh=_mesh,
    scratch_types=[
        pltpu.VMEM((NCH, CH), jnp.int32),
        pltpu.VMEM((CH, DW), jnp.float32),
        pltpu.VMEM_SHARED((R, DW), jnp.float32),
        pltpu.SemaphoreType.DMA,
    ],
    compiler_params=_sc_params,
)
def _deg_kernel(dst_hbm, zeros_hbm, ones_hbm, out_hbm, dst_v, ones_v, acc, sem):
    c = lax.axis_index("c")
    s = lax.axis_index("s")
    wid = s * NC + c
    pltpu.sync_copy(dst_hbm.at[wid], dst_v)
    pltpu.sync_copy(ones_hbm, ones_v)
    pltpu.sync_copy(zeros_hbm.at[pl.ds(s * RPT, RPT)],
                    acc.at[pl.ds(s * RPT, RPT)])
    plsc.subcore_barrier()

    def fire(j, carry):
        pltpu.async_copy(ones_v, acc.at[dst_v.at[j]], sem, add=True)
        return carry

    lax.fori_loop(0, NCH, fire, 0)

    def drain(j, carry):
        pltpu.make_async_copy(ones_v, acc.at[dst_v.at[j]], sem).wait()
        return carry

    lax.fori_loop(0, NCH, drain, 0)
    plsc.subcore_barrier()
    pltpu.sync_copy(acc.at[pl.ds(s * RPT, RPT)],
                    out_hbm.at[c, pl.ds(s * RPT, RPT)])


NBUF = 4


@functools.partial(
    pl.kernel,
    out_type=jax.ShapeDtypeStruct((NC, R, D), jnp.float32),
    mesh=_mesh,
    scratch_types=[
        pltpu.VMEM((NCH, CH), jnp.int32),
        pltpu.VMEM((NCH, CH), jnp.int32),
        [pltpu.VMEM((CH, D), jnp.float32)] * NBUF,
        pltpu.VMEM_SHARED((R, D), jnp.float32),
        pltpu.VMEM_SHARED((R, D), jnp.float32),
        [pltpu.SemaphoreType.DMA] * NBUF,
        [pltpu.SemaphoreType.DMA] * NBUF,
    ],
    compiler_params=_sc_params,
)
def _scatter_kernel(y_hbm, src_hbm, dst_hbm, out_hbm,
                    src_v, dst_v, bufs, acc, y_s, gsems, ssems):
    c = lax.axis_index("c")
    s = lax.axis_index("s")
    wid = s * NC + c
    pltpu.sync_copy(src_hbm.at[wid], src_v)
    pltpu.sync_copy(dst_hbm.at[wid], dst_v)
    pltpu.sync_copy(y_hbm.at[pl.ds(s * RPT, RPT)],
                    acc.at[pl.ds(s * RPT, RPT)])
    pltpu.sync_copy(y_hbm.at[pl.ds(s * RPT, RPT)],
                    y_s.at[pl.ds(s * RPT, RPT)])
    plsc.subcore_barrier()

    def fire_gather(j, b):
        pltpu.async_copy(y_s.at[src_v.at[j]], bufs[b], gsems[b])

    def wait_gather(j, b):
        pltpu.make_async_copy(y_s.at[src_v.at[j]], bufs[b], gsems[b]).wait()

    def fire_scatter(j, b):
        pltpu.async_copy(bufs[b], acc.at[dst_v.at[j]], ssems[b], add=True)

    def wait_scatter(j, b):
        pltpu.make_async_copy(bufs[b], acc.at[dst_v.at[j]], ssems[b]).wait()

    for b in range(NBUF):
        fire_gather(b, b)

    def body(k, carry):
        base = k * NBUF
        for b in range(NBUF):
            wait_gather(base + b, b)
            fire_scatter(base + b, b)
        for b in range(NBUF):
            wait_scatter(base + b, b)
            nj = base + NBUF + b

            @pl.when(nj < NCH)
            def _():
                fire_gather(nj, b)
        return carry

    lax.fori_loop(0, NCH // NBUF, body, 0)
    plsc.subcore_barrier()
    pltpu.sync_copy(acc.at[pl.ds(s * RPT, RPT)],
                    out_hbm.at[c, pl.ds(s * RPT, RPT)])


def _a_body(degp, x, w1, y1, dinv):
    deg = degp[0, :, 0] + degp[1, :, 0] + 1.0
    di = lax.rsqrt(deg)
    xw = jnp.dot(x[...], w1[...], preferred_element_type=jnp.float32)
    y1[...] = xw * di[:, None]
    dinv[...] = di[:, None]


def _kA(degp, x, w1):
    grid = (pl.cdiv(N, BLK),)
    return pl.pallas_call(
        _a_body,
        grid=grid,
        in_specs=[
            pl.BlockSpec((NC, BLK, DW), lambda i: (0, i, 0)),
            pl.BlockSpec((BLK, F), lambda i: (i, 0)),
            pl.BlockSpec((F, D), lambda i: (0, 0)),
        ],
        out_specs=[
            pl.BlockSpec((BLK, D), lambda i: (i, 0)),
            pl.BlockSpec((BLK, 1), lambda i: (i, 0)),
        ],
        out_shape=[
            jax.ShapeDtypeStruct((R, D), jnp.float32),
            jax.ShapeDtypeStruct((N, 1), jnp.float32),
        ],
    )(degp, x, w1)


def _b_body(p, y1, dinv, b1, w2, y2):
    accv = p[0] + p[1] - y1[...]
    h = jnp.maximum(accv * dinv[...] + b1[...], 0.0)
    xw2 = jnp.dot(h, w2[...], preferred_element_type=jnp.float32)
    y2[...] = xw2 * dinv[...]


def _kB(p1, y1, dinv, b1, w2):
    grid = (pl.cdiv(N, BLK),)
    return pl.pallas_call(
        _b_body,
        grid=grid,
        in_specs=[
            pl.BlockSpec((NC, BLK, D), lambda i: (0, i, 0)),
            pl.BlockSpec((BLK, D), lambda i: (i, 0)),
            pl.BlockSpec((BLK, 1), lambda i: (i, 0)),
            pl.BlockSpec((1, D), lambda i: (0, 0)),
            pl.BlockSpec((D, D), lambda i: (0, 0)),
        ],
        out_specs=pl.BlockSpec((BLK, D), lambda i: (i, 0)),
        out_shape=jax.ShapeDtypeStruct((R, D), jnp.float32),
    )(p1, y1, dinv, b1, w2)


def _c_body(p2, y2, dinv, b2, wmu, bmu, wlv, blv, eps, mu, lv, zr):
    z = (p2[0] + p2[1] - y2[...]) * dinv[...] + b2[...]
    m = jnp.dot(z, wmu[...], preferred_element_type=jnp.float32) + bmu[...]
    l = jnp.dot(z, wlv[...], preferred_element_type=jnp.float32) + blv[...]
    mu[...] = m
    lv[...] = l
    zr[...] = m + eps[...] * jnp.exp(0.5 * l)


def _kC(p2, y2, dinv, b2, wmu, bmu, wlv, blv, eps):
    grid = (pl.cdiv(N, BLK),)
    rowspec = pl.BlockSpec((BLK, C), lambda i: (i, 0))
    return pl.pallas_call(
        _c_body,
        grid=grid,
        in_specs=[
            pl.BlockSpec((NC, BLK, D), lambda i: (0, i, 0)),
            pl.BlockSpec((BLK, D), lambda i: (i, 0)),
            pl.BlockSpec((BLK, 1), lambda i: (i, 0)),
            pl.BlockSpec((1, D), lambda i: (0, 0)),
            pl.BlockSpec((D, C), lambda i: (0, 0)),
            pl.BlockSpec((1, C), lambda i: (0, 0)),
            pl.BlockSpec((D, C), lambda i: (0, 0)),
            pl.BlockSpec((1, C), lambda i: (0, 0)),
            rowspec,
        ],
        out_specs=[rowspec, rowspec, rowspec],
        out_shape=[
            jax.ShapeDtypeStruct((N, C), jnp.float32),
            jax.ShapeDtypeStruct((N, C), jnp.float32),
            jax.ShapeDtypeStruct((N, C), jnp.float32),
        ],
    )(p2, y2, dinv, b2, wmu, bmu, wlv, blv, eps)


BM, BN = 640, 10000


def _d_body(a, b, o):
    prod = lax.dot_general(a[...], b[...], (((1,), (1,)), ((), ())),
                           preferred_element_type=jnp.float32)
    o[...] = jax.nn.sigmoid(prod)


def _kD(zr8):
    grid = (pl.cdiv(N, BM), pl.cdiv(N, BN))
    return pl.pallas_call(
        _d_body,
        grid=grid,
        compiler_params=pltpu.CompilerParams(
            vmem_limit_bytes=114 * 1024 * 1024),
        in_specs=[
            pl.BlockSpec((BM, 8), lambda i, j: (i, 0)),
            pl.BlockSpec((BN, 8), lambda i, j: (j, 0)),
        ],
        out_specs=pl.BlockSpec((BM, BN), lambda i, j: (i, j)),
        out_shape=jax.ShapeDtypeStruct((N, N), jnp.float32),
    )(zr8, zr8)


def kernel(x, edge_index, W1, b1, W2, b2, Wmu, bmu, Wlv, blv):
    ei = edge_index.astype(jnp.int32)
    pad = E_PAD - E
    src_p = jnp.concatenate([ei[0], jnp.zeros((pad,), jnp.int32)])
    dst_p = jnp.concatenate([ei[1], jnp.full((pad,), TRASH, jnp.int32)])
    src3 = src_p.reshape(NW, NCH, CH)
    dst3 = dst_p.reshape(NW, NCH, CH)
    zeros_rw = jnp.zeros((R, DW), jnp.float32)
    ones_cw = jnp.ones((CH, DW), jnp.float32)

    degp = _deg_kernel(dst3, zeros_rw, ones_cw)
    y1, dinv = _kA(degp, x, W1)
    p1 = _scatter_kernel(y1, src3, dst3)
    y2 = _kB(p1, y1, dinv, b1.reshape(1, D), W2)
    p2 = _scatter_kernel(y2, src3, dst3)
    eps = jax.random.normal(jax.random.key(1), (N, C), jnp.float32)
    mu, logvar, zr = _kC(p2, y2, dinv, b2.reshape(1, D), Wmu,
                         bmu.reshape(1, C), Wlv, blv.reshape(1, C), eps)
    zr8 = jnp.pad(zr, ((0, 0), (0, 8 - C)))
    adj = _kD(zr8)
    return adj, mu, logvar


# speedup vs baseline: 1.0055x; 1.0055x over previous
import functools

import jax
import jax.numpy as jnp
from jax import lax
from jax.experimental import pallas as pl
from jax.experimental.pallas import tpu as pltpu
from jax.experimental.pallas import tpu_sc as plsc

N = 10000
F = 128
D = 64
C = 3
E = 160000

NC, NS, L = 2, 16, 16
NW = NC * NS
CH = 128
NCH = 40
EPT = NCH * CH
E_PAD = NW * EPT
TRASH = N
R = 10240
RPT = R // NS

BLK = 512
_mesh = plsc.VectorSubcoreMesh(core_axis_name="c", subcore_axis_name="s")
_sc_params = pltpu.CompilerParams(use_tc_tiling_on_sc=False)


DW = 16


@functools.partial(
    pl.kernel,
    out_type=jax.ShapeDtypeStruct((NC, R, DW), jnp.float32),
    mesh=_mesh,
    scratch_types=[
        pltpu.VMEM((NCH, CH), jnp.int32),
        pltpu.VMEM((CH, DW), jnp.float32),
        pltpu.VMEM_SHARED((R, DW), jnp.float32),
        pltpu.SemaphoreType.DMA,
    ],
    compiler_params=_sc_params,
)
def _deg_kernel(dst_hbm, zeros_hbm, ones_hbm, out_hbm, dst_v, ones_v, acc, sem):
    c = lax.axis_index("c")
    s = lax.axis_index("s")
    wid = s * NC + c
    pltpu.sync_copy(dst_hbm.at[wid], dst_v)
    pltpu.sync_copy(ones_hbm, ones_v)
    pltpu.sync_copy(zeros_hbm.at[pl.ds(s * RPT, RPT)],
                    acc.at[pl.ds(s * RPT, RPT)])
    plsc.subcore_barrier()

    def fire(j, carry):
        pltpu.async_copy(ones_v, acc.at[dst_v.at[j]], sem, add=True)
        return carry

    lax.fori_loop(0, NCH, fire, 0)

    def drain(j, carry):
        pltpu.make_async_copy(ones_v, acc.at[dst_v.at[j]], sem).wait()
        return carry

    lax.fori_loop(0, NCH, drain, 0)
    plsc.subcore_barrier()
    pltpu.sync_copy(acc.at[pl.ds(s * RPT, RPT)],
                    out_hbm.at[c, pl.ds(s * RPT, RPT)])


NBUF = 4


@functools.partial(
    pl.kernel,
    out_type=jax.ShapeDtypeStruct((NC, R, D), jnp.float32),
    mesh=_mesh,
    scratch_types=[
        pltpu.VMEM((NCH, CH), jnp.int32),
        pltpu.VMEM((NCH, CH), jnp.int32),
        [pltpu.VMEM((CH, D), jnp.float32)] * NBUF,
        pltpu.VMEM_SHARED((R, D), jnp.float32),
        pltpu.VMEM_SHARED((R, D), jnp.float32),
        [pltpu.SemaphoreType.DMA] * NBUF,
        [pltpu.SemaphoreType.DMA] * NBUF,
    ],
    compiler_params=_sc_params,
)
def _scatter_kernel(y_hbm, src_hbm, dst_hbm, out_hbm,
                    src_v, dst_v, bufs, acc, y_s, gsems, ssems):
    c = lax.axis_index("c")
    s = lax.axis_index("s")
    wid = s * NC + c
    pltpu.sync_copy(src_hbm.at[wid], src_v)
    pltpu.sync_copy(dst_hbm.at[wid], dst_v)
    pltpu.sync_copy(y_hbm.at[pl.ds(s * RPT, RPT)],
                    acc.at[pl.ds(s * RPT, RPT)])
    pltpu.sync_copy(y_hbm.at[pl.ds(s * RPT, RPT)],
                    y_s.at[pl.ds(s * RPT, RPT)])
    plsc.subcore_barrier()

    def fire_gather(j, b):
        pltpu.async_copy(y_s.at[src_v.at[j]], bufs[b], gsems[b])

    def wait_gather(j, b):
        pltpu.make_async_copy(y_s.at[src_v.at[j]], bufs[b], gsems[b]).wait()

    def fire_scatter(j, b):
        pltpu.async_copy(bufs[b], acc.at[dst_v.at[j]], ssems[b], add=True)

    def wait_scatter(j, b):
        pltpu.make_async_copy(bufs[b], acc.at[dst_v.at[j]], ssems[b]).wait()

    for b in range(NBUF):
        fire_gather(b, b)

    def body(k, carry):
        base = k * NBUF
        for b in range(NBUF):
            wait_gather(base + b, b)
            fire_scatter(base + b, b)
        for b in range(NBUF):
            wait_scatter(base + b, b)
            nj = base + NBUF + b

            @pl.when(nj < NCH)
            def _():
                fire_gather(nj, b)
        return carry

    lax.fori_loop(0, NCH // NBUF, body, 0)
    plsc.subcore_barrier()
    pltpu.sync_copy(acc.at[pl.ds(s * RPT, RPT)],
                    out_hbm.at[c, pl.ds(s * RPT, RPT)])


def _a_body(degp, x, w1, y1, dinv):
    deg = degp[0, :, 0] + degp[1, :, 0] + 1.0
    di = lax.rsqrt(deg)
    xw = jnp.dot(x[...], w1[...], preferred_element_type=jnp.float32)
    y1[...] = xw * di[:, None]
    dinv[...] = di[:, None]


def _kA(degp, x, w1):
    grid = (pl.cdiv(N, BLK),)
    return pl.pallas_call(
        _a_body,
        grid=grid,
        in_specs=[
            pl.BlockSpec((NC, BLK, DW), lambda i: (0, i, 0)),
            pl.BlockSpec((BLK, F), lambda i: (i, 0)),
            pl.BlockSpec((F, D), lambda i: (0, 0)),
        ],
        out_specs=[
            pl.BlockSpec((BLK, D), lambda i: (i, 0)),
            pl.BlockSpec((BLK, 1), lambda i: (i, 0)),
        ],
        out_shape=[
            jax.ShapeDtypeStruct((R, D), jnp.float32),
            jax.ShapeDtypeStruct((N, 1), jnp.float32),
        ],
    )(degp, x, w1)


def _b_body(p, y1, dinv, b1, w2, y2):
    accv = p[0] + p[1] - y1[...]
    h = jnp.maximum(accv * dinv[...] + b1[...], 0.0)
    xw2 = jnp.dot(h, w2[...], preferred_element_type=jnp.float32)
    y2[...] = xw2 * dinv[...]


def _kB(p1, y1, dinv, b1, w2):
    grid = (pl.cdiv(N, BLK),)
    return pl.pallas_call(
        _b_body,
        grid=grid,
        in_specs=[
            pl.BlockSpec((NC, BLK, D), lambda i: (0, i, 0)),
            pl.BlockSpec((BLK, D), lambda i: (i, 0)),
            pl.BlockSpec((BLK, 1), lambda i: (i, 0)),
            pl.BlockSpec((1, D), lambda i: (0, 0)),
            pl.BlockSpec((D, D), lambda i: (0, 0)),
        ],
        out_specs=pl.BlockSpec((BLK, D), lambda i: (i, 0)),
        out_shape=jax.ShapeDtypeStruct((R, D), jnp.float32),
    )(p1, y1, dinv, b1, w2)


def _c_body(p2, y2, dinv, b2, wmu, bmu, wlv, blv, eps, mu, lv, zr):
    z = (p2[0] + p2[1] - y2[...]) * dinv[...] + b2[...]
    m = jnp.dot(z, wmu[...], preferred_element_type=jnp.float32) + bmu[...]
    l = jnp.dot(z, wlv[...], preferred_element_type=jnp.float32) + blv[...]
    mu[...] = m
    lv[...] = l
    zrv = m + eps[...] * jnp.exp(0.5 * l)
    zr[...] = jnp.concatenate(
        [zrv, jnp.zeros((zrv.shape[0], 8 - C), jnp.float32)], axis=1)


def _kC(p2, y2, dinv, b2, wmu, bmu, wlv, blv, eps):
    grid = (pl.cdiv(N, BLK),)
    rowspec = pl.BlockSpec((BLK, C), lambda i: (i, 0))
    return pl.pallas_call(
        _c_body,
        grid=grid,
        in_specs=[
            pl.BlockSpec((NC, BLK, D), lambda i: (0, i, 0)),
            pl.BlockSpec((BLK, D), lambda i: (i, 0)),
            pl.BlockSpec((BLK, 1), lambda i: (i, 0)),
            pl.BlockSpec((1, D), lambda i: (0, 0)),
            pl.BlockSpec((D, C), lambda i: (0, 0)),
            pl.BlockSpec((1, C), lambda i: (0, 0)),
            pl.BlockSpec((D, C), lambda i: (0, 0)),
            pl.BlockSpec((1, C), lambda i: (0, 0)),
            rowspec,
        ],
        out_specs=[rowspec, rowspec, pl.BlockSpec((BLK, 8), lambda i: (i, 0))],
        out_shape=[
            jax.ShapeDtypeStruct((N, C), jnp.float32),
            jax.ShapeDtypeStruct((N, C), jnp.float32),
            jax.ShapeDtypeStruct((N, 8), jnp.float32),
        ],
    )(p2, y2, dinv, b2, wmu, bmu, wlv, blv, eps)


BM, BN = 640, 10000


def _d_body(a, b, o):
    prod = lax.dot_general(a[...], b[...], (((1,), (1,)), ((), ())),
                           preferred_element_type=jnp.float32)
    o[...] = jax.nn.sigmoid(prod)


def _kD(zr8):
    grid = (pl.cdiv(N, BM), pl.cdiv(N, BN))
    return pl.pallas_call(
        _d_body,
        grid=grid,
        compiler_params=pltpu.CompilerParams(
            vmem_limit_bytes=114 * 1024 * 1024),
        in_specs=[
            pl.BlockSpec((BM, 8), lambda i, j: (i, 0)),
            pl.BlockSpec((BN, 8), lambda i, j: (j, 0)),
        ],
        out_specs=pl.BlockSpec((BM, BN), lambda i, j: (i, j)),
        out_shape=jax.ShapeDtypeStruct((N, N), jnp.float32),
    )(zr8, zr8)


def kernel(x, edge_index, W1, b1, W2, b2, Wmu, bmu, Wlv, blv):
    ei = edge_index.astype(jnp.int32)
    pad = E_PAD - E
    src_p = jnp.concatenate([ei[0], jnp.zeros((pad,), jnp.int32)])
    dst_p = jnp.concatenate([ei[1], jnp.full((pad,), TRASH, jnp.int32)])
    src3 = src_p.reshape(NW, NCH, CH)
    dst3 = dst_p.reshape(NW, NCH, CH)
    zeros_rw = jnp.zeros((R, DW), jnp.float32)
    ones_cw = jnp.ones((CH, DW), jnp.float32)

    degp = _deg_kernel(dst3, zeros_rw, ones_cw)
    y1, dinv = _kA(degp, x, W1)
    p1 = _scatter_kernel(y1, src3, dst3)
    y2 = _kB(p1, y1, dinv, b1.reshape(1, D), W2)
    p2 = _scatter_kernel(y2, src3, dst3)
    eps = jax.random.normal(jax.random.key(1), (N, C), jnp.float32)
    mu, logvar, zr8 = _kC(p2, y2, dinv, b2.reshape(1, D), Wmu,
                          bmu.reshape(1, C), Wlv, blv.reshape(1, C), eps)
    adj = _kD(zr8)
    return adj, mu, logvar
